# baseline (speedup 1.0000x reference)
import jax
import jax.numpy as jnp
from jax.experimental import pallas as pl


def _copy_body(x_ref, o_ref):
    o_ref[...] = x_ref[...]


def kernel(stack):
    B = 8
    n = stack.shape[0] // B
    return pl.pallas_call(
        _copy_body,
        grid=(n,),
        in_specs=[pl.BlockSpec((B, 1024, 64), lambda i: (i, 0, 0))],
        out_specs=pl.BlockSpec((B, 1024, 64), lambda i: (i, 0, 0)),
        out_shape=jax.ShapeDtypeStruct(stack.shape, stack.dtype),
    )(stack)
